# Initial kernel scaffold; baseline (speedup 1.0000x reference)
import jax
import jax.numpy as jnp
from jax.experimental import pallas as pl


def kernel(x, edge_index, W1_0, b1_0, W2_0, b2_0, W1_1, b1_1, W2_1, b2_1, W1_2, b1_2, W2_2, b2_2):
    raise NotImplementedError("write your pallas kernel here")


# speedup vs baseline: 6.1687x; 6.1687x over previous
import functools

import jax
import jax.numpy as jnp
from jax import lax
from jax.experimental import pallas as pl
from jax.experimental.pallas import tpu as pltpu
from jax.experimental.pallas import tpu_sc as plsc

N = 10000
E = 320000
NC = 2
NS = 16
NW = NC * NS
EPW = E // NW
CHUNK = 80
NCH = EPW // CHUNK
RPT = 632
NP = RPT * NS

ROW_BLK = 1000


def _make_segsum(W):
  mesh = plsc.VectorSubcoreMesh(
      core_axis_name="c", subcore_axis_name="s",
      num_cores=NC, num_subcores=NS)

  @functools.partial(
      pl.kernel,
      out_type=jax.ShapeDtypeStruct((NC, NP, W), jnp.float32),
      mesh=mesh,
      compiler_params=pltpu.CompilerParams(use_tc_tiling_on_sc=(W == 128)),
      scratch_types=[
          pltpu.VMEM((NCH, CHUNK), jnp.int32),
          pltpu.VMEM((NCH, CHUNK), jnp.int32),
          pltpu.VMEM((CHUNK, W), jnp.float32),
          pltpu.VMEM_SHARED((NP, W), jnp.float32),
          pltpu.SemaphoreType.DMA,
      ],
  )
  def segsum(g_hbm, src_hbm, dst_hbm, zeros_hbm, out_hbm, src_v, dst_v, gbuf,
             acc, sem):
    c = lax.axis_index("c")
    s = lax.axis_index("s")
    wid = c * NS + s

    pltpu.sync_copy(src_hbm.at[wid], src_v)
    pltpu.sync_copy(dst_hbm.at[wid], dst_v)

    pltpu.sync_copy(zeros_hbm, acc.at[pl.ds(s * RPT, RPT), :])
    plsc.subcore_barrier()

    def chunk_body(j, _):
      pltpu.async_copy(g_hbm.at[src_v.at[j]], gbuf, sem).wait()
      pltpu.sync_copy(gbuf, acc.at[dst_v.at[j]], add=True)
      return 0

    lax.fori_loop(0, NCH, chunk_body, 0)
    plsc.subcore_barrier()

    pltpu.sync_copy(acc.at[pl.ds(s * RPT, RPT), :],
                    out_hbm.at[c, pl.ds(s * RPT, RPT), :])

  return segsum


_segsum_128 = _make_segsum(128)
_segsum_64 = _make_segsum(64)

_P = lax.Precision.HIGHEST


def _mm_body(x_ref, w_ref, o_ref):
  o_ref[...] = jnp.dot(x_ref[...], w_ref[...],
                       preferred_element_type=jnp.float32, precision=_P)


def _mid_body(g_ref, sa_ref, sb_ref, b1_ref, w2_ref, b2_ref, w1n_ref, o_ref):
  z = jnp.maximum(g_ref[...] + sa_ref[...] + sb_ref[...] + b1_ref[...], 0.0)
  h = jnp.dot(z, w2_ref[...], preferred_element_type=jnp.float32,
              precision=_P) + b2_ref[...]
  h = jnp.maximum(h, 0.0)
  o_ref[...] = jnp.dot(h, w1n_ref[...], preferred_element_type=jnp.float32,
                       precision=_P)


def _final_body(g_ref, sa_ref, sb_ref, b1_ref, w2_ref, b2_ref, o_ref):
  z = jnp.maximum(g_ref[...] + sa_ref[...] + sb_ref[...] + b1_ref[...], 0.0)
  o_ref[...] = jnp.dot(z, w2_ref[...], preferred_element_type=jnp.float32,
                       precision=_P) + b2_ref[...]


def _row_spec(w):
  return pl.BlockSpec((ROW_BLK, w), lambda i: (i, 0))


def _full_spec(r, w):
  return pl.BlockSpec((r, w), lambda i: (0, 0))


def _mm(x, w):
  n, k = x.shape
  m = w.shape[1]
  return pl.pallas_call(
      _mm_body,
      grid=(n // ROW_BLK,),
      in_specs=[_row_spec(k), _full_spec(k, m)],
      out_specs=_row_spec(m),
      out_shape=jax.ShapeDtypeStruct((n, m), jnp.float32),
  )(x, w)


def _mid(g, sa, sb, b1, w2, b2, w1n):
  n, k = g.shape
  m = w1n.shape[1]
  return pl.pallas_call(
      _mid_body,
      grid=(n // ROW_BLK,),
      in_specs=[_row_spec(k), _row_spec(k), _row_spec(k),
                _full_spec(1, k), _full_spec(k, k), _full_spec(1, k),
                _full_spec(k, m)],
      out_specs=_row_spec(m),
      out_shape=jax.ShapeDtypeStruct((n, m), jnp.float32),
  )(g, sa, sb, b1, w2, b2, w1n)


def _final(g, sa, sb, b1, w2, b2):
  n, k = g.shape
  m = w2.shape[1]
  return pl.pallas_call(
      _final_body,
      grid=(n // ROW_BLK,),
      in_specs=[_row_spec(k), _row_spec(k), _row_spec(k),
                _full_spec(1, k), _full_spec(k, m), _full_spec(1, m)],
      out_specs=_row_spec(m),
      out_shape=jax.ShapeDtypeStruct((n, m), jnp.float32),
  )(g, sa, sb, b1, w2, b2)


def kernel(x, edge_index, W1_0, b1_0, W2_0, b2_0, W1_1, b1_1, W2_1, b2_1,
           W1_2, b1_2, W2_2, b2_2):
  src = edge_index[0].astype(jnp.int32).reshape(NW, NCH, CHUNK)
  dst = edge_index[1].astype(jnp.int32).reshape(NW, NCH, CHUNK)

  W1_2p = jnp.pad(W1_2, ((0, 0), (0, 24)))
  b1_2p = jnp.pad(b1_2, (0, 24)).reshape(1, 64)
  W2_2p = jnp.pad(W2_2, ((0, 24), (0, 0)))

  b1_0r = b1_0.reshape(1, 128)
  b2_0r = b2_0.reshape(1, 128)
  b1_1r = b1_1.reshape(1, 128)
  b2_1r = b2_1.reshape(1, 128)
  b2_2r = b2_2.reshape(1, 40)

  z128 = jnp.zeros((RPT, 128), jnp.float32)
  z64 = jnp.zeros((RPT, 64), jnp.float32)

  g0 = _mm(x, W1_0)
  s0 = _segsum_128(g0, src, dst, z128)
  g1 = _mid(g0, s0[0], s0[1], b1_0r, W2_0, b2_0r, W1_1)
  s1 = _segsum_128(g1, src, dst, z128)
  g2 = _mid(g1, s1[0], s1[1], b1_1r, W2_1, b2_1r, W1_2p)
  s2 = _segsum_64(g2, src, dst, z64)
  return _final(g2, s2[0], s2[1], b1_2p, W2_2p, b2_2r)
